# baseline (speedup 1.0000x reference)
import functools
import math

import jax
import jax.numpy as jnp
from jax import lax
from jax.experimental import pallas as pl
from jax.experimental.pallas import tpu as pltpu
from jax.experimental.pallas import tpu_sc as plsc

H = 4096
G = 16
K = 4
S = 8192

_INV_SQRT2 = 1.0 / math.sqrt(2.0)


def _gelu(x):
    return x * 0.5 * (1.0 + lax.erf(x * _INV_SQRT2))


def _mean_rel_body(x_ref, emb_ref, mean_ref, rel_ref, acc_ref):
    i = pl.program_id(0)

    @pl.when(i == 0)
    def _init():
        acc_ref[...] = jnp.zeros_like(acc_ref)

    ones = jnp.ones((1, x_ref.shape[0]), jnp.float32)
    acc_ref[...] += jnp.dot(ones, x_ref[...],
                            preferred_element_type=jnp.float32)

    @pl.when(i == pl.num_programs(0) - 1)
    def _fin():
        m = acc_ref[...] * (1.0 / S)
        mean_ref[...] = m
        rel_ref[...] = jnp.sum(emb_ref[...] * m, axis=1, keepdims=True)


def _mean_and_relevance(x2d, emb):
    blk = 512
    grid = (S // blk,)
    return pl.pallas_call(
        _mean_rel_body,
        grid=grid,
        in_specs=[
            pl.BlockSpec((blk, H), lambda i: (i, 0)),
            pl.BlockSpec((G, H), lambda i: (0, 0)),
        ],
        out_specs=[
            pl.BlockSpec((1, H), lambda i: (0, 0)),
            pl.BlockSpec((G, 1), lambda i: (0, 0)),
        ],
        out_shape=[
            jax.ShapeDtypeStruct((1, H), jnp.float32),
            jax.ShapeDtypeStruct((G, 1), jnp.float32),
        ],
        scratch_shapes=[pltpu.VMEM((1, H), jnp.float32)],
    )(x2d, emb)


def _sc_topk(rel, prio):
    mesh = plsc.VectorSubcoreMesh(core_axis_name="c", subcore_axis_name="s")

    @functools.partial(
        pl.kernel,
        out_type=jax.ShapeDtypeStruct((2 * G,), jnp.float32),
        mesh=mesh,
        compiler_params=pltpu.CompilerParams(needs_layout_passes=False),
        scratch_types=[
            pltpu.VMEM((G,), jnp.float32),
            pltpu.VMEM((G,), jnp.int32),
            pltpu.VMEM((G,), jnp.float32),
            pltpu.VMEM((2 * G,), jnp.float32),
            pltpu.SemaphoreType.DMA,
        ],
    )
    def k(rel_hbm, prio_hbm, packed_out,
          rel_v, idx_v, prio_v, packed_v, sem):
        c = lax.axis_index("c")
        s = lax.axis_index("s")

        @pl.when(jnp.logical_and(c == 0, s == 0))
        def _():
            pltpu.sync_copy(rel_hbm, rel_v)
            pltpu.sync_copy(prio_hbm, prio_v)
            keys = rel_v[...]
            lanes = lax.iota(jnp.int32, G)
            lanes_f = lanes.astype(jnp.float32)
            one = jnp.ones((G,), jnp.int32)
            zero = jnp.zeros((G,), jnp.int32)
            rank = jnp.zeros((G,), jnp.int32)
            for j in range(1, G):
                idx_rot = (lanes + j) & (G - 1)
                kj = plsc.load_gather(rel_v, [idx_rot])
                gt = (kj > keys) | ((kj == keys) & (idx_rot < lanes))
                rank = rank + jnp.where(gt, one, zero)
            plsc.store_scatter(idx_v, [rank], lanes)
            plsc.store_scatter(packed_v, [rank], lanes_f)
            pg = plsc.load_gather(prio_v, [idx_v[...]])
            packed_v[pl.ds(G, G)] = pg
            pltpu.sync_copy(packed_v, packed_out)

    return k(rel, prio)


def _btop_body(m_ref, w_ref, hm_ref, acc_ref):
    i = pl.program_id(0)

    @pl.when(i == 0)
    def _init():
        acc_ref[...] = jnp.zeros_like(acc_ref)

    acc_ref[...] += jnp.dot(m_ref[...], w_ref[...],
                            preferred_element_type=jnp.float32)

    @pl.when(i == pl.num_programs(0) - 1)
    def _fin():
        hm_ref[...] = acc_ref[...]


def _btop(mean_row, w1):
    blk = 512
    grid = (H // blk,)
    return pl.pallas_call(
        _btop_body,
        grid=grid,
        in_specs=[
            pl.BlockSpec((1, blk), lambda t: (0, t)),
            pl.BlockSpec((blk, H), lambda t: (t, 0)),
        ],
        out_specs=pl.BlockSpec((1, H), lambda t: (0, 0)),
        out_shape=jax.ShapeDtypeStruct((1, H), jnp.float32),
        scratch_shapes=[pltpu.VMEM((1, H), jnp.float32)],
    )(mean_row, w1)


_P1 = H // 512
_P2 = H // 512
_E = H // 2


def _fused_body(rel_ref, emb_ref, hm_ref, w1_ref, b1_ref, g_ref, beta_ref,
                w2_ref, b2_ref, we1_ref, be1_ref, we2_ref, be2_ref,
                plans_ref, prog_ref, acc_ref, sel_ref, hs_ref, eacc_ref):
    t = pl.program_id(0)

    @pl.when(t == 0)
    def _sel():
        relv = rel_ref[...]
        subio = lax.broadcasted_iota(jnp.int32, (G, 1), 0).astype(jnp.float32)
        rowio = lax.broadcasted_iota(jnp.int32, (8, G), 0)
        lanio = lax.broadcasted_iota(jnp.int32, (8, G), 1).astype(jnp.float32)
        onehot = jnp.zeros((8, G), jnp.float32)
        for r in range(8):
            m = jnp.max(relv, axis=0, keepdims=True)
            cand = jnp.where(relv == m, subio, float(G))
            fi = jnp.min(cand, axis=0, keepdims=True)
            onehot = onehot + jnp.where(
                (rowio == r) & (lanio == fi), 1.0, 0.0)
            relv = jnp.where(subio == fi, -3.4e38, relv)
        sel_ref[...] = jnp.dot(onehot, emb_ref[...],
                               preferred_element_type=jnp.float32)
        acc_ref[...] = jnp.zeros_like(acc_ref)

    @pl.when(t < _P1)
    def _p1():
        c = t
        acc_ref[...] += jnp.dot(sel_ref[:, pl.ds(c * 512, 512)], w1_ref[...],
                                preferred_element_type=jnp.float32)

        @pl.when(t == _P1 - 1)
        def _fin1():
            hv = acc_ref[...] + hm_ref[...] + b1_ref[...]
            mu = jnp.mean(hv, axis=-1, keepdims=True)
            var = jnp.mean((hv - mu) ** 2, axis=-1, keepdims=True)
            normed = ((hv - mu) * lax.rsqrt(var + 1e-5) * g_ref[...]
                      + beta_ref[...])
            hs_ref[...] = _gelu(normed)
            eacc_ref[...] = jnp.zeros_like(eacc_ref)

    @pl.when(t >= _P1)
    def _p2():
        c = t - _P1
        pblk = jnp.dot(hs_ref[...], w2_ref[...],
                       preferred_element_type=jnp.float32) + b2_ref[...]
        plans_ref[:, pl.ds(c * 512, 512)] = pblk
        eacc_ref[...] += jnp.dot(pblk, we1_ref[...],
                                 preferred_element_type=jnp.float32)

        @pl.when(t == _P1 + _P2 - 1)
        def _fin2():
            e = _gelu(eacc_ref[...] + be1_ref[...])
            prog_ref[...] = jax.nn.sigmoid(
                jnp.sum(e * we2_ref[...], axis=1, keepdims=True)
                + be2_ref[...])


def _fused(rel_col, emb, hm, w1, b1, g, beta, w2, b2, we1, be1, we2_row, be2):
    grid = (_P1 + _P2,)
    c2 = lambda t: (0, jnp.clip(t - _P1, 0, _P2 - 1))
    r2 = lambda t: (jnp.clip(t - _P1, 0, _P2 - 1), 0)
    return pl.pallas_call(
        _fused_body,
        grid=grid,
        in_specs=[
            pl.BlockSpec((G, 1), lambda t: (0, 0)),
            pl.BlockSpec((G, H), lambda t: (0, 0)),
            pl.BlockSpec((1, H), lambda t: (0, 0)),
            pl.BlockSpec((512, H),
                         lambda t: (jnp.minimum(t, _P1 - 1) + _P1, 0)),
            pl.BlockSpec((1, H), lambda t: (0, 0)),
            pl.BlockSpec((1, H), lambda t: (0, 0)),
            pl.BlockSpec((1, H), lambda t: (0, 0)),
            pl.BlockSpec((H, 512), c2),
            pl.BlockSpec((1, 512), c2),
            pl.BlockSpec((512, _E), r2),
            pl.BlockSpec((1, _E), lambda t: (0, 0)),
            pl.BlockSpec((1, _E), lambda t: (0, 0)),
            pl.BlockSpec((1, 1), lambda t: (0, 0)),
        ],
        out_specs=[
            pl.BlockSpec((8, H), lambda t: (0, 0)),
            pl.BlockSpec((8, 1), lambda t: (0, 0)),
        ],
        out_shape=[
            jax.ShapeDtypeStruct((8, H), jnp.float32),
            jax.ShapeDtypeStruct((8, 1), jnp.float32),
        ],
        scratch_shapes=[
            pltpu.VMEM((8, H), jnp.float32),
            pltpu.VMEM((8, H), jnp.float32),
            pltpu.VMEM((8, H), jnp.float32),
            pltpu.VMEM((8, _E), jnp.float32),
        ],
    )(rel_col, emb, hm, w1, b1, g, beta, w2, b2, we1, be1, we2_row, be2)


def kernel(current_state, desired_outcome, goal_embeddings, goal_priorities,
           W1, b1, ln_gamma, ln_beta, W2, b2, We1, be1, We2, be2):
    x2d = current_state.reshape(S, H)
    mean_state, rel = _mean_and_relevance(x2d, goal_embeddings)

    packed = _sc_topk(rel.reshape(G), goal_priorities)
    idx0 = packed[:K].astype(jnp.int32)
    priorities = packed[G:G + K]

    hm = _btop(mean_state, W1)

    plans8, prog8 = _fused(
        rel, goal_embeddings, hm, W1, b1.reshape(1, H),
        ln_gamma.reshape(1, H), ln_beta.reshape(1, H), W2, b2.reshape(1, H),
        We1, be1.reshape(1, H // 2), We2.reshape(1, H // 2),
        be2.reshape(1, 1))

    return (idx0, plans8[:K], prog8[:K], priorities)


# speedup vs baseline: 1.0112x; 1.0112x over previous
import functools
import math

import jax
import jax.numpy as jnp
from jax import lax
from jax.experimental import pallas as pl
from jax.experimental.pallas import tpu as pltpu
from jax.experimental.pallas import tpu_sc as plsc

H = 4096
G = 16
K = 4
S = 8192

_INV_SQRT2 = 1.0 / math.sqrt(2.0)


def _gelu(x):
    return x * 0.5 * (1.0 + lax.erf(x * _INV_SQRT2))


def _mean_rel_body(x_ref, emb_ref, mean_ref, rel_ref, acc_ref):
    i = pl.program_id(0)

    @pl.when(i == 0)
    def _init():
        acc_ref[...] = jnp.zeros_like(acc_ref)

    ones = jnp.ones((1, x_ref.shape[0]), jnp.float32)
    acc_ref[...] += jnp.dot(ones, x_ref[...],
                            preferred_element_type=jnp.float32)

    @pl.when(i == pl.num_programs(0) - 1)
    def _fin():
        m = acc_ref[...] * (1.0 / S)
        mean_ref[...] = m
        rel_ref[...] = jnp.sum(emb_ref[...] * m, axis=1, keepdims=True)


def _mean_and_relevance(x2d, emb):
    blk = 512
    grid = (S // blk,)
    return pl.pallas_call(
        _mean_rel_body,
        grid=grid,
        in_specs=[
            pl.BlockSpec((blk, H), lambda i: (i, 0)),
            pl.BlockSpec((G, H), lambda i: (0, 0)),
        ],
        out_specs=[
            pl.BlockSpec((1, H), lambda i: (0, 0)),
            pl.BlockSpec((G, 1), lambda i: (0, 0)),
        ],
        out_shape=[
            jax.ShapeDtypeStruct((1, H), jnp.float32),
            jax.ShapeDtypeStruct((G, 1), jnp.float32),
        ],
        scratch_shapes=[pltpu.VMEM((1, H), jnp.float32)],
    )(x2d, emb)


def _sc_topk(rel, prio):
    mesh = plsc.VectorSubcoreMesh(core_axis_name="c", subcore_axis_name="s")

    @functools.partial(
        pl.kernel,
        out_type=(
            jax.ShapeDtypeStruct((8, 1), jnp.float32),
            jax.ShapeDtypeStruct((2 * G,), jnp.float32),
        ),
        mesh=mesh,
        compiler_params=pltpu.CompilerParams(needs_layout_passes=False),
        scratch_types=[
            pltpu.VMEM((G,), jnp.float32),
            pltpu.VMEM((G,), jnp.int32),
            pltpu.VMEM((8, 1), jnp.float32),
            pltpu.VMEM((G,), jnp.float32),
            pltpu.VMEM((2 * G,), jnp.float32),
            pltpu.SemaphoreType.DMA,
        ],
    )
    def k(rel_hbm, prio_hbm, idx8_out, packed_out,
          rel_v, idx_v, idx8_v, prio_v, packed_v, sem):
        c = lax.axis_index("c")
        s = lax.axis_index("s")

        @pl.when(jnp.logical_and(c == 0, s == 0))
        def _():
            pltpu.sync_copy(rel_hbm, rel_v)
            pltpu.sync_copy(prio_hbm, prio_v)
            keys = rel_v[...]
            lanes = lax.iota(jnp.int32, G)
            lanes_f = lanes.astype(jnp.float32)
            one = jnp.ones((G,), jnp.int32)
            zero = jnp.zeros((G,), jnp.int32)
            rank = jnp.zeros((G,), jnp.int32)
            for j in range(1, G):
                idx_rot = (lanes + j) & (G - 1)
                kj = plsc.load_gather(rel_v, [idx_rot])
                gt = (kj > keys) | ((kj == keys) & (idx_rot < lanes))
                rank = rank + jnp.where(gt, one, zero)
            plsc.store_scatter(idx_v, [rank], lanes)
            plsc.store_scatter(idx8_v, [rank, zero], lanes_f,
                              mask=rank < 8)
            plsc.store_scatter(packed_v, [rank], lanes_f)
            pg = plsc.load_gather(prio_v, [idx_v[...]])
            packed_v[pl.ds(G, G)] = pg
            pltpu.sync_copy(idx8_v, idx8_out)
            pltpu.sync_copy(packed_v, packed_out)

    return k(rel, prio)


def _btop_body(m_ref, w_ref, hm_ref, acc_ref):
    i = pl.program_id(0)

    @pl.when(i == 0)
    def _init():
        acc_ref[...] = jnp.zeros_like(acc_ref)

    acc_ref[...] += jnp.dot(m_ref[...], w_ref[...],
                            preferred_element_type=jnp.float32)

    @pl.when(i == pl.num_programs(0) - 1)
    def _fin():
        hm_ref[...] = acc_ref[...]


def _btop(mean_row, w1):
    blk = 512
    grid = (H // blk,)
    return pl.pallas_call(
        _btop_body,
        grid=grid,
        in_specs=[
            pl.BlockSpec((1, blk), lambda t: (0, t)),
            pl.BlockSpec((blk, H), lambda t: (t, 0)),
        ],
        out_specs=pl.BlockSpec((1, H), lambda t: (0, 0)),
        out_shape=jax.ShapeDtypeStruct((1, H), jnp.float32),
        scratch_shapes=[pltpu.VMEM((1, H), jnp.float32)],
    )(mean_row, w1)


_P1 = H // 512
_P2 = H // 512
_E = H // 2


def _fused_body(idx_ref, emb_ref, hm_ref, w1_ref, b1_ref, g_ref, beta_ref,
                w2_ref, b2_ref, we1_ref, be1_ref, we2_ref, be2_ref,
                plans_ref, prog_ref, acc_ref, sel_ref, hs_ref, eacc_ref):
    t = pl.program_id(0)

    @pl.when(t == 0)
    def _sel():
        gidx = lax.broadcasted_iota(jnp.int32, (8, G), 1).astype(jnp.float32)
        onehot = jnp.where(gidx == idx_ref[...], 1.0, 0.0)
        sel_ref[...] = jnp.dot(onehot, emb_ref[...],
                               preferred_element_type=jnp.float32)
        acc_ref[...] = jnp.zeros_like(acc_ref)

    @pl.when(t < _P1)
    def _p1():
        c = t
        acc_ref[...] += jnp.dot(sel_ref[:, pl.ds(c * 512, 512)], w1_ref[...],
                                preferred_element_type=jnp.float32)

        @pl.when(t == _P1 - 1)
        def _fin1():
            hv = acc_ref[...] + hm_ref[...] + b1_ref[...]
            mu = jnp.mean(hv, axis=-1, keepdims=True)
            var = jnp.mean((hv - mu) ** 2, axis=-1, keepdims=True)
            normed = ((hv - mu) * lax.rsqrt(var + 1e-5) * g_ref[...]
                      + beta_ref[...])
            hs_ref[...] = _gelu(normed)
            eacc_ref[...] = jnp.zeros_like(eacc_ref)

    @pl.when(t >= _P1)
    def _p2():
        c = t - _P1
        pblk = jnp.dot(hs_ref[...], w2_ref[...],
                       preferred_element_type=jnp.float32) + b2_ref[...]
        plans_ref[:, pl.ds(c * 512, 512)] = pblk
        eacc_ref[...] += jnp.dot(pblk, we1_ref[...],
                                 preferred_element_type=jnp.float32)

        @pl.when(t == _P1 + _P2 - 1)
        def _fin2():
            e = _gelu(eacc_ref[...] + be1_ref[...])
            prog_ref[...] = jax.nn.sigmoid(
                jnp.sum(e * we2_ref[...], axis=1, keepdims=True)
                + be2_ref[...])


def _fused(idx8f, emb, hm, w1, b1, g, beta, w2, b2, we1, be1, we2_row, be2):
    grid = (_P1 + _P2,)
    c2 = lambda t: (0, jnp.clip(t - _P1, 0, _P2 - 1))
    r2 = lambda t: (jnp.clip(t - _P1, 0, _P2 - 1), 0)
    return pl.pallas_call(
        _fused_body,
        grid=grid,
        in_specs=[
            pl.BlockSpec((8, 1), lambda t: (0, 0)),
            pl.BlockSpec((G, H), lambda t: (0, 0)),
            pl.BlockSpec((1, H), lambda t: (0, 0)),
            pl.BlockSpec((512, H),
                         lambda t: (jnp.minimum(t, _P1 - 1) + _P1, 0)),
            pl.BlockSpec((1, H), lambda t: (0, 0)),
            pl.BlockSpec((1, H), lambda t: (0, 0)),
            pl.BlockSpec((1, H), lambda t: (0, 0)),
            pl.BlockSpec((H, 512), c2),
            pl.BlockSpec((1, 512), c2),
            pl.BlockSpec((512, _E), r2),
            pl.BlockSpec((1, _E), lambda t: (0, 0)),
            pl.BlockSpec((1, _E), lambda t: (0, 0)),
            pl.BlockSpec((1, 1), lambda t: (0, 0)),
        ],
        out_specs=[
            pl.BlockSpec((8, H), lambda t: (0, 0)),
            pl.BlockSpec((8, 1), lambda t: (0, 0)),
        ],
        out_shape=[
            jax.ShapeDtypeStruct((8, H), jnp.float32),
            jax.ShapeDtypeStruct((8, 1), jnp.float32),
        ],
        scratch_shapes=[
            pltpu.VMEM((8, H), jnp.float32),
            pltpu.VMEM((8, H), jnp.float32),
            pltpu.VMEM((8, H), jnp.float32),
            pltpu.VMEM((8, _E), jnp.float32),
        ],
    )(idx8f, emb, hm, w1, b1, g, beta, w2, b2, we1, be1, we2_row, be2)


def kernel(current_state, desired_outcome, goal_embeddings, goal_priorities,
           W1, b1, ln_gamma, ln_beta, W2, b2, We1, be1, We2, be2):
    x2d = current_state.reshape(S, H)
    mean_state, rel = _mean_and_relevance(x2d, goal_embeddings)

    idx8f, packed = _sc_topk(rel.reshape(G), goal_priorities)
    idx0 = packed[:K].astype(jnp.int32)
    priorities = packed[G:G + K]

    hm = _btop(mean_state, W1)

    plans8, prog8 = _fused(
        idx8f, goal_embeddings, hm, W1, b1.reshape(1, H),
        ln_gamma.reshape(1, H), ln_beta.reshape(1, H), W2, b2.reshape(1, H),
        We1, be1.reshape(1, H // 2), We2.reshape(1, H // 2),
        be2.reshape(1, 1))

    return (idx0, plans8[:K], prog8[:K], priorities)
